# baseline (speedup 1.0000x reference)
import functools

import jax
import jax.numpy as jnp
from jax import lax
from jax.experimental import pallas as pl
from jax.experimental.pallas import tpu as pltpu
from jax.experimental.pallas import tpu_sc as plsc

_NUM_CORES = 2
_NUM_SUBCORES = 16
_NUM_WORKERS = _NUM_CORES * _NUM_SUBCORES

_HDIM = 16
_CHUNK = 2048


def _sc_gather(table, idx):
    vocab, dim = table.shape
    samples_per_row = dim // _HDIM
    table_flat = table.reshape(vocab * samples_per_row, _HDIM)
    off = (
        idx[:, None] * samples_per_row
        + jnp.arange(samples_per_row, dtype=jnp.int32)[None, :]
    ).reshape(-1)
    n = off.shape[0]
    per_w = n // _NUM_WORKERS
    n_chunks = per_w // _CHUNK
    assert per_w % _CHUNK == 0, (n, per_w)
    idx3 = off.reshape(n // _CHUNK, _CHUNK)

    mesh = plsc.VectorSubcoreMesh(core_axis_name="c", subcore_axis_name="s")

    @functools.partial(
        pl.kernel,
        out_type=jax.ShapeDtypeStruct((n // _CHUNK, _CHUNK, _HDIM), jnp.float32),
        mesh=mesh,
        scratch_types=[
            pltpu.VMEM((2, _CHUNK), jnp.int32),
            pltpu.VMEM((2, _CHUNK, _HDIM), jnp.float32),
            pltpu.SemaphoreType.DMA,
        ],
        compiler_params=pltpu.CompilerParams(use_tc_tiling_on_sc=False),
    )
    def k(table_hbm, idx_hbm, out_hbm, idx_v, rows_v, gsem):
        wid = lax.axis_index("s") * _NUM_CORES + lax.axis_index("c")
        chunk_base = wid * n_chunks

        def load_idx(g, buf):
            pltpu.sync_copy(idx_hbm.at[chunk_base + g], idx_v.at[buf])

        def start_gather(buf):
            return pltpu.async_copy(
                table_hbm.at[idx_v.at[buf]], rows_v.at[buf], gsem
            )

        def store_out(g, buf):
            pltpu.sync_copy(
                rows_v.at[buf],
                out_hbm.at[chunk_base + g],
            )

        load_idx(0, 0)
        pending = start_gather(0)
        for g in range(1, n_chunks):
            buf = g % 2
            load_idx(g, buf)
            nxt = start_gather(buf)
            pending.wait()
            store_out(g - 1, 1 - buf)
            pending = nxt
        pending.wait()
        store_out(n_chunks - 1, (n_chunks - 1) % 2)

    return k(table_flat, idx3)


def _tc_assemble(x_num, wmat, bflat, cat2d):
    b, n_num = x_num.shape
    ncd = cat2d.shape[1]
    nnd = wmat.shape[1]
    blk = 1024
    grid = (b // blk,)

    def body(x_ref, w_ref, b_ref, cat_ref, out_ref):
        num = (
            jnp.dot(x_ref[...], w_ref[...], preferred_element_type=jnp.float32)
            + b_ref[...]
        )
        out_ref[:, :nnd] = num
        out_ref[:, nnd:] = cat_ref[...]

    return pl.pallas_call(
        body,
        grid=grid,
        in_specs=[
            pl.BlockSpec((blk, n_num), lambda i: (i, 0)),
            pl.BlockSpec((n_num, nnd), lambda i: (0, 0)),
            pl.BlockSpec((1, nnd), lambda i: (0, 0)),
            pl.BlockSpec((blk, ncd), lambda i: (i, 0)),
        ],
        out_specs=pl.BlockSpec((blk, nnd + ncd), lambda i: (i, 0)),
        out_shape=jax.ShapeDtypeStruct((b, nnd + ncd), jnp.float32),
    )(x_num, wmat, bflat, cat2d)


def kernel(x_num, x_cat, cat_table, num_weight, num_bias):
    b, n_cat = x_cat.shape
    n_num = x_num.shape[1]
    dim = cat_table.shape[1]
    idx = x_cat.astype(jnp.int32).reshape(-1)
    cat2d = _sc_gather(cat_table, idx).reshape(b, n_cat * dim)
    feat = jnp.arange(n_num * dim, dtype=jnp.int32) // dim
    mask = feat[None, :] == jnp.arange(n_num, dtype=jnp.int32)[:, None]
    wmat = jnp.where(mask, num_weight.reshape(-1)[None, :], 0.0)
    bflat = num_bias.reshape(1, n_num * dim)
    out = _tc_assemble(x_num, wmat, bflat, cat2d)
    return out.reshape(b, n_num + n_cat, dim)


# speedup vs baseline: 1.0212x; 1.0212x over previous
import functools

import jax
import jax.numpy as jnp
from jax import lax
from jax.experimental import pallas as pl
from jax.experimental.pallas import tpu as pltpu
from jax.experimental.pallas import tpu_sc as plsc

_NUM_CORES = 2
_NUM_SUBCORES = 16
_NUM_WORKERS = _NUM_CORES * _NUM_SUBCORES

_CHUNK = 1024


def _sc_gather_t(table, idx_t, n_num, n_cat):
    vocab, dim = table.shape
    b = idx_t.shape[1]
    n_tok = n_num + n_cat
    assert dim == 32 and b % _CHUNK == 0
    chunks_per_tok = b // _CHUNK
    n_chunks = n_cat * chunks_per_tok
    assert n_chunks % _NUM_WORKERS == 0
    per_w = n_chunks // _NUM_WORKERS

    mesh = plsc.VectorSubcoreMesh(core_axis_name="c", subcore_axis_name="s")

    @functools.partial(
        pl.kernel,
        out_type=jax.ShapeDtypeStruct(
            (n_tok, dim // 8, (b // 128) * 8 * 128), jnp.float32
        ),
        mesh=mesh,
        scratch_types=[
            pltpu.VMEM((2, _CHUNK), jnp.int32),
            pltpu.VMEM((2, _CHUNK, dim), jnp.float32),
            pltpu.VMEM((dim // 8, _CHUNK * 8), jnp.float32),
            pltpu.SemaphoreType.DMA,
        ],
        compiler_params=pltpu.CompilerParams(
            use_tc_tiling_on_sc=False, needs_layout_passes=False
        ),
    )
    def k(table_hbm, idx_hbm, out_hbm, idx_v, rows_v, tr_v, gsem):
        wid = lax.axis_index("s") * _NUM_CORES + lax.axis_index("c")
        chunk_base = wid * per_w

        def load_idx(c, buf):
            f, bc = c // chunks_per_tok, c % chunks_per_tok
            pltpu.sync_copy(
                idx_hbm.at[f, pl.ds(bc * _CHUNK, _CHUNK)], idx_v.at[buf]
            )

        def start_gather(buf):
            return pltpu.async_copy(
                table_hbm.at[idx_v.at[buf]], rows_v.at[buf], gsem
            )

        def transpose_store(c, buf):
            rows = rows_v.at[buf]
            lane = lax.iota(jnp.int32, 16)

            @plsc.parallel_loop(0, _CHUNK // 16, unroll=1)
            def grp_body(g):
                row_idx = g * 16 + lane
                inner_base = ((g >> 3) * 1024 + (g & 7) * 16) + lane
                for d in range(32):
                    vals = plsc.load_gather(
                        rows, [row_idx, jnp.full((16,), d, jnp.int32)]
                    )
                    plsc.store_scatter(
                        tr_v,
                        [
                            jnp.full((16,), d // 8, jnp.int32),
                            inner_base + (d % 8) * 128,
                        ],
                        vals,
                    )
            f, bc = c // chunks_per_tok, c % chunks_per_tok
            pltpu.sync_copy(
                tr_v,
                out_hbm.at[n_num + f, :, pl.ds(bc * _CHUNK * 8, _CHUNK * 8)],
            )

        load_idx(chunk_base, 0)
        pending = start_gather(0)
        for j in range(1, per_w):
            buf = j % 2
            load_idx(chunk_base + j, buf)
            nxt = start_gather(buf)
            pending.wait()
            transpose_store(chunk_base + j - 1, 1 - buf)
            pending = nxt
        pending.wait()
        transpose_store(chunk_base + per_w - 1, (per_w - 1) % 2)

    return k(table, idx_t)


def _tc_num_fill(z, x3, w3, b3):
    n_tok, dh, bh, dl, bl = z.shape
    n_num = x3.shape[0]
    blk = 8
    grid = (bh // blk,)

    def body(z_ref, x_ref, w_ref, b_ref, out_ref):
        x = x_ref[...][:, None, :, None, :]
        w = w_ref[...][:, :, None, :, None]
        bb = b_ref[...][:, :, None, :, None]
        out_ref[...] = x * w + bb

    return pl.pallas_call(
        body,
        grid=grid,
        in_specs=[
            pl.BlockSpec(memory_space=pl.ANY),
            pl.BlockSpec((n_num, blk, bl), lambda i: (0, i, 0)),
            pl.BlockSpec((n_num, dh, dl), lambda i: (0, 0, 0)),
            pl.BlockSpec((n_num, dh, dl), lambda i: (0, 0, 0)),
        ],
        out_specs=pl.BlockSpec(
            (n_num, dh, blk, dl, bl), lambda i: (0, 0, i, 0, 0)
        ),
        out_shape=jax.ShapeDtypeStruct(z.shape, jnp.float32),
        input_output_aliases={0: 0},
    )(z, x3, w3, b3)


def kernel(x_num, x_cat, cat_table, num_weight, num_bias):
    b, n_cat = x_cat.shape
    n_num = x_num.shape[1]
    dim = cat_table.shape[1]
    idx_t = jnp.pad(x_cat.T.astype(jnp.int32), ((0, (-n_cat) % 8), (0, 0)))
    z = _sc_gather_t(cat_table, idx_t, n_num, n_cat)
    z5 = z.reshape(n_num + n_cat, dim // 8, b // 128, 8, 128)
    x3 = x_num.T.reshape(n_num, b // 128, 128)
    w3 = num_weight.reshape(n_num, dim // 8, 8)
    b3 = num_bias.reshape(n_num, dim // 8, 8)
    z5 = _tc_num_fill(z5, x3, w3, b3)
    e = z5.transpose(2, 4, 0, 1, 3).reshape(b, n_num + n_cat, dim)
    return e
